# baseline (speedup 1.0000x reference)
import functools

import jax
import jax.numpy as jnp
from jax import lax
from jax.experimental import pallas as pl
from jax.experimental.pallas import tpu as pltpu
from jax.experimental.pallas import tpu_sc as plsc

_B, _N, _D, _K = 8, 2048, 128, 16
_NW = 32
_TOT_F = _B * _N * _K
_TOT_E = _B * _N * _K * _K
_C = 128
_GF = 1
_GE = 16

_F_ROWS = _TOT_F // _C
_E_ROWS = _TOT_E // _C
_F_RPT = _F_ROWS // _NW
_E_RPT = _E_ROWS // _NW

_mesh = plsc.VectorSubcoreMesh(core_axis_name="c", subcore_axis_name="s")


@functools.partial(
    pl.kernel,
    out_type=(
        jax.ShapeDtypeStruct((_TOT_F, _D), jnp.float32),
        jax.ShapeDtypeStruct((_E_ROWS, _C), jnp.float32),
        jax.ShapeDtypeStruct((_E_ROWS, _C), jnp.float32),
    ),
    mesh=_mesh,
    scratch_types=(
        pltpu.VMEM((2, _GF, _C), jnp.int32),
        pltpu.VMEM((2, _GF * _C, _D), jnp.float32),
        pltpu.VMEM((2, _GE, _C), jnp.int32),
        pltpu.VMEM((2, _GE, _C), jnp.float32),
        pltpu.VMEM((2, _GE, _C), jnp.float32),
        pltpu.SemaphoreType.DMA,
        pltpu.SemaphoreType.DMA,
        pltpu.SemaphoreType.DMA,
        pltpu.SemaphoreType.DMA,
        pltpu.SemaphoreType.DMA,
        pltpu.SemaphoreType.DMA,
        pltpu.SemaphoreType.DMA,
        pltpu.SemaphoreType.DMA,
        pltpu.SemaphoreType.DMA,
        pltpu.SemaphoreType.DMA,
        pltpu.SemaphoreType.DMA,
        pltpu.SemaphoreType.DMA,
    ),
)
def _gather_kernel(feat_hbm, adj_hbm, edges_hbm, fidx_hbm, eidx_hbm,
                   wf_hbm, wa_hbm, we_hbm,
                   fidx_v, frows_v, eidx_v, eadj_v, eedg_v,
                   sem_ge0, sem_ge1, sem_oe0, sem_oe1,
                   sem_gf0, sem_gf1, sem_of0, sem_of1,
                   sem_ie0, sem_ie1, sem_if0, sem_if1):
    wid = lax.axis_index("s") * 2 + lax.axis_index("c")
    sem_ge = (sem_ge0, sem_ge1)
    sem_oe = (sem_oe0, sem_oe1)
    sem_gf = (sem_gf0, sem_gf1)
    sem_of = (sem_of0, sem_of1)
    sem_ie = (sem_ie0, sem_ie1)
    sem_if = (sem_if0, sem_if1)

    class _Multi:
        def __init__(self, cs):
            self.cs = cs

        def start(self):
            for c in self.cs:
                c.start()

        def wait(self):
            for c in self.cs:
                c.wait()

    f_row0 = wid * _F_RPT

    def f_idx(slot, step):
        return pltpu.make_async_copy(
            fidx_hbm.at[pl.ds(f_row0 + step * _GF, _GF)],
            fidx_v.at[slot], sem_if[slot])

    def f_fire(slot, step):
        f_idx(slot, step).wait()
        for j in range(_GF):
            pltpu.async_copy(feat_hbm.at[fidx_v.at[slot, j]],
                             frows_v.at[slot, pl.ds(j * _C, _C)],
                             sem_gf[slot])

    def f_wait_g(slot):
        for j in range(_GF):
            pltpu.make_async_copy(feat_hbm.at[fidx_v.at[slot, j]],
                                  frows_v.at[slot, pl.ds(j * _C, _C)],
                                  sem_gf[slot]).wait()

    def f_out(slot, step):
        return pltpu.make_async_copy(
            frows_v.at[slot],
            wf_hbm.at[pl.ds((f_row0 + step * _GF) * _C, _GF * _C)],
            sem_of[slot])

    e_row0 = wid * _E_RPT

    def e_gathers(slot):
        cs = []
        for j in range(_GE):
            idx_row = eidx_v.at[slot, j]
            cs.append(pltpu.make_async_copy(
                adj_hbm.at[idx_row], eadj_v.at[slot, j], sem_ge[slot]))
            cs.append(pltpu.make_async_copy(
                edges_hbm.at[idx_row], eedg_v.at[slot, j], sem_ge[slot]))
        return cs

    def e_idx(slot, step):
        return pltpu.make_async_copy(
            eidx_hbm.at[pl.ds(e_row0 + step * _GE, _GE)],
            eidx_v.at[slot], sem_ie[slot])

    def e_fire(slot, step):
        e_idx(slot, step).wait()
        for c in e_gathers(slot):
            c.start()

    def e_wait_g(slot):
        for c in e_gathers(slot):
            c.wait()

    def e_out(slot, step):
        dst = pl.ds(e_row0 + step * _GE, _GE)
        return _Multi([
            pltpu.make_async_copy(eadj_v.at[slot], wa_hbm.at[dst],
                                  sem_oe[slot]),
            pltpu.make_async_copy(eedg_v.at[slot], we_hbm.at[dst],
                                  sem_oe[slot])])

    n_e = _E_RPT // _GE
    n_f = _F_RPT // _GF
    assert n_e == n_f
    n_body = n_e // 2

    e_idx(0, 0).start()
    e_fire(0, 0)
    f_idx(0, 0).start()
    f_fire(0, 0)
    e_idx(1, 1).start()
    f_idx(1, 1).start()

    def body(t, carry):
        pl.when(t > 0)(lambda: e_out(1, 2 * t - 1).wait())
        e_fire(1, 2 * t + 1)
        pl.when(t > 0)(lambda: f_out(1, 2 * t - 1).wait())
        f_fire(1, 2 * t + 1)
        e_wait_g(0)
        pl.when(t < n_body - 1)(lambda: e_idx(0, 2 * t + 2).start())
        e_out(0, 2 * t).start()
        f_wait_g(0)
        pl.when(t < n_body - 1)(lambda: f_idx(0, 2 * t + 2).start())
        f_out(0, 2 * t).start()

        def refill_e():
            e_out(0, 2 * t).wait()
            e_fire(0, 2 * t + 2)

        def refill_f():
            f_out(0, 2 * t).wait()
            f_fire(0, 2 * t + 2)

        pl.when(t < n_body - 1)(refill_e)
        pl.when(t < n_body - 1)(refill_f)
        e_wait_g(1)
        pl.when(t < n_body - 1)(lambda: e_idx(1, 2 * t + 3).start())
        e_out(1, 2 * t + 1).start()
        f_wait_g(1)
        pl.when(t < n_body - 1)(lambda: f_idx(1, 2 * t + 3).start())
        f_out(1, 2 * t + 1).start()
        return carry

    lax.fori_loop(0, n_body, body, 0)
    e_out(0, n_e - 2).wait()
    e_out(1, n_e - 1).wait()
    f_out(0, n_f - 2).wait()
    f_out(1, n_f - 1).wait()


def _tiled_flat(x):
    return x.reshape(_B, _N // 8, 8, _N // 128, 128) \
            .transpose(0, 1, 3, 2, 4).reshape(_B * _N * _N)


def kernel(features, adj_matrix, edges_matrix, all_neighbours):
    nb = all_neighbours.astype(jnp.int32)
    boff = jnp.arange(_B, dtype=jnp.int32) * (_N * _N)
    fb = jnp.arange(_B, dtype=jnp.int32) * _N
    fidx = (fb[:, None, None] + nb[None]).reshape(_F_ROWS, _C)
    rowpart = (nb >> 3) * (8 * _N) + (nb & 7) * 128
    colpart = (nb >> 7) * 1024 + (nb & 127)
    rp = rowpart.T.reshape(_K, _N // 128, 128)
    cp = colpart.T.reshape(2, 8, _N // 128, 128).transpose(0, 2, 1, 3)
    eidx = (boff[:, None, None, None, None, None]
            + rp[None, :, None, :, None, :]
            + cp[None, None, :, :, :, :]).reshape(_E_ROWS, _C)
    wf, wa, we = _gather_kernel(
        features.reshape(_B * _N, _D),
        _tiled_flat(adj_matrix),
        _tiled_flat(edges_matrix),
        fidx, eidx)

    def _devectorize(buf):
        return buf.reshape(_B, _K, 2, _N // 128, 8, 128) \
                  .transpose(0, 3, 5, 1, 2, 4).reshape(_B, _N, _K, _K)

    return (wf.reshape(_B, _N, _K, _D), _devectorize(wa), _devectorize(we))


# speedup vs baseline: 1.0151x; 1.0151x over previous
import functools

import jax
import jax.numpy as jnp
from jax import lax
from jax.experimental import pallas as pl
from jax.experimental.pallas import tpu as pltpu
from jax.experimental.pallas import tpu_sc as plsc

_B, _N, _D, _K = 8, 2048, 128, 16
_NW = 32
_TOT_F = _B * _N * _K
_TOT_E = _B * _N * _K * _K
_C = 128
_GF = 1
_GE = 16

_F_ROWS = _TOT_F // _C
_E_ROWS = _TOT_E // _C
_F_RPT = _F_ROWS // _NW
_E_RPT = _E_ROWS // _NW

_mesh = plsc.VectorSubcoreMesh(core_axis_name="c", subcore_axis_name="s")


@functools.partial(
    pl.kernel,
    out_type=(
        jax.ShapeDtypeStruct((_TOT_F, _D), jnp.float32),
        jax.ShapeDtypeStruct((_E_ROWS, _C), jnp.float32),
        jax.ShapeDtypeStruct((_E_ROWS, _C), jnp.float32),
    ),
    mesh=_mesh,
    scratch_types=(
        pltpu.VMEM((2, _GF, _C), jnp.int32),
        pltpu.VMEM((2, _GF * _C, _D), jnp.float32),
        pltpu.VMEM((2, _GE, _C), jnp.int32),
        pltpu.VMEM((2, _GE, _C), jnp.float32),
        pltpu.VMEM((2, _GE, _C), jnp.float32),
        pltpu.SemaphoreType.DMA,
        pltpu.SemaphoreType.DMA,
        pltpu.SemaphoreType.DMA,
        pltpu.SemaphoreType.DMA,
        pltpu.SemaphoreType.DMA,
        pltpu.SemaphoreType.DMA,
        pltpu.SemaphoreType.DMA,
        pltpu.SemaphoreType.DMA,
        pltpu.SemaphoreType.DMA,
        pltpu.SemaphoreType.DMA,
        pltpu.SemaphoreType.DMA,
        pltpu.SemaphoreType.DMA,
    ),
)
def _gather_kernel(feat_hbm, adj_hbm, edges_hbm, fidx_hbm, eidx_hbm,
                   wf_hbm, wa_hbm, we_hbm,
                   fidx_v, frows_v, eidx_v, eadj_v, eedg_v,
                   sem_ge0, sem_ge1, sem_oe0, sem_oe1,
                   sem_gf0, sem_gf1, sem_of0, sem_of1,
                   sem_ie0, sem_ie1, sem_if0, sem_if1):
    wid = lax.axis_index("s") * 2 + lax.axis_index("c")
    sem_ge = (sem_ge0, sem_ge1)
    sem_oe = (sem_oe0, sem_oe1)
    sem_gf = (sem_gf0, sem_gf1)
    sem_of = (sem_of0, sem_of1)
    sem_ie = (sem_ie0, sem_ie1)
    sem_if = (sem_if0, sem_if1)

    class _Multi:
        def __init__(self, cs):
            self.cs = cs

        def start(self):
            for c in self.cs:
                c.start()

        def wait(self):
            for c in self.cs:
                c.wait()

    f_row0 = wid * _F_RPT

    def f_idx(slot, step):
        return pltpu.make_async_copy(
            fidx_hbm.at[pl.ds(f_row0 + step * _GF, _GF)],
            fidx_v.at[slot], sem_if[slot])

    def f_fire(slot, step):
        f_idx(slot, step).wait()
        for j in range(_GF):
            pltpu.async_copy(feat_hbm.at[fidx_v.at[slot, j]],
                             frows_v.at[slot, pl.ds(j * _C, _C)],
                             sem_gf[slot])

    def f_wait_g(slot):
        for j in range(_GF):
            pltpu.make_async_copy(feat_hbm.at[fidx_v.at[slot, j]],
                                  frows_v.at[slot, pl.ds(j * _C, _C)],
                                  sem_gf[slot]).wait()

    def f_out(slot, step):
        return pltpu.make_async_copy(
            frows_v.at[slot],
            wf_hbm.at[pl.ds((f_row0 + step * _GF) * _C, _GF * _C)],
            sem_of[slot])

    e_row0 = wid * _E_RPT
    e_nb_row0 = (wid % 4) * _E_RPT
    b_off = (wid // 4) * (_N * _N)

    def e_gathers(slot):
        cs = []
        for j in range(_GE):
            idx_row = eidx_v.at[slot, j]
            cs.append(pltpu.make_async_copy(
                adj_hbm.at[idx_row], eadj_v.at[slot, j], sem_ge[slot]))
            cs.append(pltpu.make_async_copy(
                edges_hbm.at[idx_row], eedg_v.at[slot, j], sem_ge[slot]))
        return cs

    def e_idx(slot, step):
        return pltpu.make_async_copy(
            eidx_hbm.at[pl.ds(e_nb_row0 + step * _GE, _GE)],
            eidx_v.at[slot], sem_ie[slot])

    def e_fire(slot, step):
        e_idx(slot, step).wait()
        for j in range(_GE):
            for k in range(_C // 16):
                sl = pl.ds(k * 16, 16)
                eidx_v[slot, j, sl] = eidx_v[slot, j, sl] + b_off
        for c in e_gathers(slot):
            c.start()

    def e_wait_g(slot):
        for c in e_gathers(slot):
            c.wait()

    def e_out(slot, step):
        dst = pl.ds(e_row0 + step * _GE, _GE)
        return _Multi([
            pltpu.make_async_copy(eadj_v.at[slot], wa_hbm.at[dst],
                                  sem_oe[slot]),
            pltpu.make_async_copy(eedg_v.at[slot], we_hbm.at[dst],
                                  sem_oe[slot])])

    n_e = _E_RPT // _GE
    n_f = _F_RPT // _GF
    assert n_e == n_f
    n_body = n_e // 2

    e_idx(0, 0).start()
    e_fire(0, 0)
    f_idx(0, 0).start()
    f_fire(0, 0)
    e_idx(1, 1).start()
    f_idx(1, 1).start()

    def body(t, carry):
        pl.when(t > 0)(lambda: e_out(1, 2 * t - 1).wait())
        e_fire(1, 2 * t + 1)
        pl.when(t > 0)(lambda: f_out(1, 2 * t - 1).wait())
        f_fire(1, 2 * t + 1)
        e_wait_g(0)
        pl.when(t < n_body - 1)(lambda: e_idx(0, 2 * t + 2).start())
        e_out(0, 2 * t).start()
        f_wait_g(0)
        pl.when(t < n_body - 1)(lambda: f_idx(0, 2 * t + 2).start())
        f_out(0, 2 * t).start()

        def refill_e():
            e_out(0, 2 * t).wait()
            e_fire(0, 2 * t + 2)

        def refill_f():
            f_out(0, 2 * t).wait()
            f_fire(0, 2 * t + 2)

        pl.when(t < n_body - 1)(refill_e)
        pl.when(t < n_body - 1)(refill_f)
        e_wait_g(1)
        pl.when(t < n_body - 1)(lambda: e_idx(1, 2 * t + 3).start())
        e_out(1, 2 * t + 1).start()
        f_wait_g(1)
        pl.when(t < n_body - 1)(lambda: f_idx(1, 2 * t + 3).start())
        f_out(1, 2 * t + 1).start()
        return carry

    lax.fori_loop(0, n_body, body, 0)
    e_out(0, n_e - 2).wait()
    e_out(1, n_e - 1).wait()
    f_out(0, n_f - 2).wait()
    f_out(1, n_f - 1).wait()


def _tiled_flat(x):
    return x.reshape(_B, _N // 8, 8, _N // 128, 128) \
            .transpose(0, 1, 3, 2, 4).reshape(_B * _N * _N)


def kernel(features, adj_matrix, edges_matrix, all_neighbours):
    nb = all_neighbours.astype(jnp.int32)
    fb = jnp.arange(_B, dtype=jnp.int32) * _N
    fidx = (fb[:, None, None] + nb[None]).reshape(_F_ROWS, _C)
    rowpart = (nb >> 3) * (8 * _N) + (nb & 7) * 128
    colpart = (nb >> 7) * 1024 + (nb & 127)
    rp = rowpart.T.reshape(_K, _N // 128, 128)
    cp = colpart.T.reshape(2, 8, _N // 128, 128).transpose(0, 2, 1, 3)
    eidx = (rp[:, None, :, None, :]
            + cp[None, :, :, :, :]).reshape(_E_ROWS // _B, _C)
    wf, wa, we = _gather_kernel(
        features.reshape(_B * _N, _D),
        _tiled_flat(adj_matrix),
        _tiled_flat(edges_matrix),
        fidx, eidx)

    def _devectorize(buf):
        return buf.reshape(_B, _K, 2, _N // 128, 8, 128) \
                  .transpose(0, 3, 5, 1, 2, 4).reshape(_B, _N, _K, _K)

    return (wf.reshape(_B, _N, _K, _D), _devectorize(wa), _devectorize(we))
